# baseline (speedup 1.0000x reference)
import functools

import jax
import jax.numpy as jnp
from jax import lax
from jax.experimental import pallas as pl
from jax.experimental.pallas import tpu as pltpu
from jax.experimental.pallas import tpu_sc as plsc

NUM_NODES = 10000
NUM_EDGES = 320000
NUM_RELATIONS = 1000
HIDDEN = 128

NC = 2
NS = 16
L = 16
NW = NC * NS
EPW = NUM_EDGES // NW
B = 80
NCH = EPW // B
NG = B // L
U = 32


def _norm_body(z_ref, o_ref):
    x = z_ref[...]
    n = jnp.sum(jnp.abs(x), axis=1, keepdims=True)
    o_ref[...] = x / jnp.maximum(n, 1e-12)


def _l1_normalize_rows(z):
    return pl.pallas_call(
        _norm_body,
        out_shape=jax.ShapeDtypeStruct((NUM_NODES, HIDDEN), jnp.float32),
        grid=(5,),
        in_specs=[pl.BlockSpec((NUM_NODES // 5, HIDDEN), lambda i: (i, 0))],
        out_specs=pl.BlockSpec((NUM_NODES // 5, HIDDEN), lambda i: (i, 0)),
    )(z)


def _sc_body(znorm_hbm, rel_hbm, hidx_hbm, tidx_hbm, ridx_hbm, out_hbm,
             hidx_v, tidx_v, ridx_v,
             h0, t0, r0, h1, t1, r1, out_v, s0, s1):
    wid = lax.axis_index("s") * NC + lax.axis_index("c")
    pltpu.sync_copy(hidx_hbm.at[pl.ds(wid * EPW, EPW)], hidx_v)
    pltpu.sync_copy(tidx_hbm.at[pl.ds(wid * EPW, EPW)], tidx_v)
    pltpu.sync_copy(ridx_hbm.at[pl.ds(wid * EPW, EPW)], ridx_v)

    row16 = lax.iota(jnp.int32, L)

    def issue(i, hb, tb, rb, sem):
        pltpu.async_copy(znorm_hbm.at[hidx_v.at[pl.ds(i * B, B)]], hb, sem)
        pltpu.async_copy(znorm_hbm.at[tidx_v.at[pl.ds(i * B, B)]], tb, sem)
        pltpu.async_copy(rel_hbm.at[ridx_v.at[pl.ds(i * B, B)]], rb, sem)

    def drain(hb, tb, rb, sem):
        pltpu.make_async_copy(znorm_hbm.at[pl.ds(0, B)], hb, sem).wait()
        pltpu.make_async_copy(znorm_hbm.at[pl.ds(0, B)], tb, sem).wait()
        pltpu.make_async_copy(rel_hbm.at[pl.ds(0, B)], rb, sem).wait()

    zero16 = jnp.zeros((L,), jnp.int32)

    def compute(i, hb, tb, rb):
        for g in range(NG):
            flat0 = (row16 + (g * L)) * HIDDEN
            z16f = jnp.zeros((L,), jnp.float32)

            def dblk(s, carry):
                a0, a1, flat = carry
                for u in range(U):
                    cu = flat + u
                    h = plsc.load_gather(hb, [zero16, cu])
                    t = plsc.load_gather(tb, [zero16, cu])
                    r = plsc.load_gather(rb, [zero16, cu])
                    v = jnp.abs(h + r - t)
                    if u % 2 == 0:
                        a0 = a0 + v
                    else:
                        a1 = a1 + v
                return a0, a1, flat + U

            a0, a1, _ = lax.fori_loop(0, HIDDEN // U, dblk,
                                      (z16f, z16f, flat0))
            out_v[pl.ds(i * B + g * L, L)] = -(a0 + a1)

    issue(0, h0, t0, r0, s0)

    def pair(k, _):
        i = k * 2
        issue(i + 1, h1, t1, r1, s1)
        drain(h0, t0, r0, s0)
        compute(i, h0, t0, r0)
        issue(i + 2, h0, t0, r0, s0)
        drain(h1, t1, r1, s1)
        compute(i + 1, h1, t1, r1)
        return 0

    lax.fori_loop(0, (NCH - 1) // 2, pair, 0)
    drain(h0, t0, r0, s0)
    compute(NCH - 1, h0, t0, r0)
    pltpu.sync_copy(out_v, out_hbm.at[pl.ds(wid * EPW, EPW)])


@jax.jit
def _sc_score(znorm, rel_emb, hidx, tidx, ridx):
    mesh = plsc.VectorSubcoreMesh(core_axis_name="c", subcore_axis_name="s",
                                  num_cores=NC, num_subcores=NS)
    return pl.kernel(
        _sc_body,
        out_type=jax.ShapeDtypeStruct((NUM_EDGES,), jnp.float32),
        mesh=mesh,
        compiler_params=pltpu.CompilerParams(needs_layout_passes=False,
                                             disable_bounds_checks=True),
        scratch_types=[
            pltpu.VMEM((EPW,), jnp.int32),
            pltpu.VMEM((EPW,), jnp.int32),
            pltpu.VMEM((EPW,), jnp.int32),
            pltpu.VMEM((B, HIDDEN), jnp.float32),
            pltpu.VMEM((B, HIDDEN), jnp.float32),
            pltpu.VMEM((B, HIDDEN), jnp.float32),
            pltpu.VMEM((B, HIDDEN), jnp.float32),
            pltpu.VMEM((B, HIDDEN), jnp.float32),
            pltpu.VMEM((B, HIDDEN), jnp.float32),
            pltpu.VMEM((EPW,), jnp.float32),
            pltpu.SemaphoreType.DMA,
            pltpu.SemaphoreType.DMA,
        ],
    )(znorm, rel_emb, hidx, tidx, ridx)


def kernel(z, edge_index, edge_type, rel_emb):
    znorm = _l1_normalize_rows(z)
    hidx = edge_index[0].astype(jnp.int32)
    tidx = edge_index[1].astype(jnp.int32)
    ridx = edge_type.astype(jnp.int32)
    return _sc_score(znorm, rel_emb, hidx, tidx, ridx)


# speedup vs baseline: 3.3441x; 3.3441x over previous
import functools

import jax
import jax.numpy as jnp
from jax import lax
from jax.experimental import pallas as pl
from jax.experimental.pallas import tpu as pltpu
from jax.experimental.pallas import tpu_sc as plsc

NUM_NODES = 10000
NUM_EDGES = 320000
NUM_RELATIONS = 1000
HIDDEN = 128

NC = 2
NS = 16
L = 16
NW = NC * NS
EPW = NUM_EDGES // NW
B = 80
NCH = EPW // B
NG = B // L
U = 32


def _norm_body(z_ref, o_ref):
    x = z_ref[...]
    n = jnp.sum(jnp.abs(x), axis=1, keepdims=True)
    o_ref[...] = x / jnp.maximum(n, 1e-12)


def _l1_normalize_rows(z):
    return pl.pallas_call(
        _norm_body,
        out_shape=jax.ShapeDtypeStruct((NUM_NODES, HIDDEN), jnp.float32),
        grid=(5,),
        in_specs=[pl.BlockSpec((NUM_NODES // 5, HIDDEN), lambda i: (i, 0))],
        out_specs=pl.BlockSpec((NUM_NODES // 5, HIDDEN), lambda i: (i, 0)),
    )(z)


def _sc_body(znorm_hbm, rel_hbm, hidx_hbm, tidx_hbm, ridx_hbm, out_hbm,
             hidx_v, tidx_v, ridx_v,
             h0, t0, r0, h1, t1, r1, out_v, s0, s1):
    wid = lax.axis_index("s") * NC + lax.axis_index("c")
    pltpu.sync_copy(hidx_hbm.at[pl.ds(wid * EPW, EPW)], hidx_v)
    pltpu.sync_copy(tidx_hbm.at[pl.ds(wid * EPW, EPW)], tidx_v)
    pltpu.sync_copy(ridx_hbm.at[pl.ds(wid * EPW, EPW)], ridx_v)

    row16 = lax.iota(jnp.int32, L)

    def issue(i, hb, tb, rb, sem):
        pltpu.async_copy(znorm_hbm.at[hidx_v.at[pl.ds(i * B, B)]], hb, sem)
        pltpu.async_copy(znorm_hbm.at[tidx_v.at[pl.ds(i * B, B)]], tb, sem)
        pltpu.async_copy(rel_hbm.at[ridx_v.at[pl.ds(i * B, B)]], rb, sem)

    def drain(hb, tb, rb, sem):
        pltpu.make_async_copy(znorm_hbm.at[pl.ds(0, B)], hb, sem).wait()
        pltpu.make_async_copy(znorm_hbm.at[pl.ds(0, B)], tb, sem).wait()
        pltpu.make_async_copy(rel_hbm.at[pl.ds(0, B)], rb, sem).wait()

    def perm(v, m):
        return v.at[row16 ^ m].get(mode="promise_in_bounds")

    def combine(a, b, m):
        sa = a + perm(a, m)
        sb = b + perm(b, m)
        return jnp.where((row16 & m) == 0, sa, perm(sb, m))

    def compute(i, hb, tb, rb):
        def group(g, _):
            ps = []
            for j in range(L):
                e = g * L + j
                vs = []
                for k in range(HIDDEN // L):
                    h = hb[e, pl.ds(k * L, L)]
                    t = tb[e, pl.ds(k * L, L)]
                    r = rb[e, pl.ds(k * L, L)]
                    vs.append(jnp.abs(h + r - t))
                while len(vs) > 1:
                    vs = [vs[a] + vs[a + 1] for a in range(0, len(vs), 2)]
                ps.append(vs[0])
            m = 1
            while len(ps) > 1:
                ps = [combine(ps[a], ps[a + 1], m)
                      for a in range(0, len(ps), 2)]
                m *= 2
            out_v[pl.ds(i * B + g * L, L)] = -ps[0]
            return 0

        lax.fori_loop(0, NG, group, 0)

    issue(0, h0, t0, r0, s0)

    def pair(k, _):
        i = k * 2
        issue(i + 1, h1, t1, r1, s1)
        drain(h0, t0, r0, s0)
        compute(i, h0, t0, r0)
        issue(i + 2, h0, t0, r0, s0)
        drain(h1, t1, r1, s1)
        compute(i + 1, h1, t1, r1)
        return 0

    lax.fori_loop(0, (NCH - 1) // 2, pair, 0)
    drain(h0, t0, r0, s0)
    compute(NCH - 1, h0, t0, r0)
    pltpu.sync_copy(out_v, out_hbm.at[pl.ds(wid * EPW, EPW)])


@jax.jit
def _sc_score(znorm, rel_emb, hidx, tidx, ridx):
    mesh = plsc.VectorSubcoreMesh(core_axis_name="c", subcore_axis_name="s",
                                  num_cores=NC, num_subcores=NS)
    return pl.kernel(
        _sc_body,
        out_type=jax.ShapeDtypeStruct((NUM_EDGES,), jnp.float32),
        mesh=mesh,
        compiler_params=pltpu.CompilerParams(needs_layout_passes=False,
                                             disable_bounds_checks=True),
        scratch_types=[
            pltpu.VMEM((EPW,), jnp.int32),
            pltpu.VMEM((EPW,), jnp.int32),
            pltpu.VMEM((EPW,), jnp.int32),
            pltpu.VMEM((B, HIDDEN), jnp.float32),
            pltpu.VMEM((B, HIDDEN), jnp.float32),
            pltpu.VMEM((B, HIDDEN), jnp.float32),
            pltpu.VMEM((B, HIDDEN), jnp.float32),
            pltpu.VMEM((B, HIDDEN), jnp.float32),
            pltpu.VMEM((B, HIDDEN), jnp.float32),
            pltpu.VMEM((EPW,), jnp.float32),
            pltpu.SemaphoreType.DMA,
            pltpu.SemaphoreType.DMA,
        ],
    )(znorm, rel_emb, hidx, tidx, ridx)


def kernel(z, edge_index, edge_type, rel_emb):
    znorm = _l1_normalize_rows(z)
    hidx = edge_index[0].astype(jnp.int32)
    tidx = edge_index[1].astype(jnp.int32)
    ridx = edge_type.astype(jnp.int32)
    return _sc_score(znorm, rel_emb, hidx, tidx, ridx)


# speedup vs baseline: 6.2015x; 1.8544x over previous
import jax
import jax.numpy as jnp
from jax import lax
from jax.experimental import pallas as pl
from jax.experimental.pallas import tpu as pltpu
from jax.experimental.pallas import tpu_sc as plsc

NUM_NODES = 10000
NUM_EDGES = 320000
NUM_RELATIONS = 1000
HIDDEN = 128

NC = 2
NS = 16
L = 16
NW = NC * NS
EPW = NUM_EDGES // NW
B = 80
NCH = EPW // B
NG = B // L
HP = HIDDEN // 2


def _norm_body(z_ref, o_ref):
    x = z_ref[...]
    n = jnp.sum(jnp.abs(x), axis=1, keepdims=True)
    o_ref[...] = (x / jnp.maximum(n, 1e-12)).astype(jnp.bfloat16)


def _l1_normalize_rows_bf16(z):
    return pl.pallas_call(
        _norm_body,
        out_shape=jax.ShapeDtypeStruct((NUM_NODES, HIDDEN), jnp.bfloat16),
        grid=(5,),
        in_specs=[pl.BlockSpec((NUM_NODES // 5, HIDDEN), lambda i: (i, 0))],
        out_specs=pl.BlockSpec((NUM_NODES // 5, HIDDEN), lambda i: (i, 0)),
    )(z)


def _pack_pairs(x_bf16):
    n, d = x_bf16.shape
    return lax.bitcast_convert_type(x_bf16.reshape(n, d // 2, 2), jnp.float32)


def _sc_body(znorm_hbm, rel_hbm, hidx_hbm, tidx_hbm, ridx_hbm, out_hbm,
             hidx_v, tidx_v, ridx_v,
             h0, t0, r0, h1, t1, r1, out_v, s0, s1):
    wid = lax.axis_index("s") * NC + lax.axis_index("c")
    pltpu.sync_copy(hidx_hbm.at[pl.ds(wid * EPW, EPW)], hidx_v)
    pltpu.sync_copy(tidx_hbm.at[pl.ds(wid * EPW, EPW)], tidx_v)
    pltpu.sync_copy(ridx_hbm.at[pl.ds(wid * EPW, EPW)], ridx_v)

    row16 = lax.iota(jnp.int32, L)

    def issue(i, hb, tb, rb, sem):
        pltpu.async_copy(znorm_hbm.at[hidx_v.at[pl.ds(i * B, B)]], hb, sem)
        pltpu.async_copy(znorm_hbm.at[tidx_v.at[pl.ds(i * B, B)]], tb, sem)
        pltpu.async_copy(rel_hbm.at[ridx_v.at[pl.ds(i * B, B)]], rb, sem)

    def drain(hb, tb, rb, sem):
        pltpu.make_async_copy(znorm_hbm.at[pl.ds(0, B)], hb, sem).wait()
        pltpu.make_async_copy(znorm_hbm.at[pl.ds(0, B)], tb, sem).wait()
        pltpu.make_async_copy(rel_hbm.at[pl.ds(0, B)], rb, sem).wait()

    def perm(v, m):
        return v.at[row16 ^ m].get(mode="promise_in_bounds")

    def combine(a, b, m):
        sa = a + perm(a, m)
        sb = b + perm(b, m)
        return jnp.where((row16 & m) == 0, sa, perm(sb, m))

    def compute(i, hb, tb, rb):
        def group(g, _):
            ps = []
            for j in range(L):
                e = g * L + j
                acc_a = jnp.zeros((L,), jnp.float32)
                acc_b = jnp.zeros((L,), jnp.float32)
                for k in range(HP // L):
                    h = plsc.bitcast(hb[e, pl.ds(k * L, L)], jnp.bfloat16)
                    t = plsc.bitcast(tb[e, pl.ds(k * L, L)], jnp.bfloat16)
                    r = plsc.bitcast(rb[e, pl.ds(k * L, L)], jnp.bfloat16)
                    v = jnp.abs(h + r - t)
                    va, vb = plsc.unpack(v, format=plsc.PackFormat.INTERLEAVED)
                    acc_a = acc_a + va
                    acc_b = acc_b + vb
                ps.append(acc_a + acc_b)
            m = 1
            while len(ps) > 1:
                ps = [combine(ps[a], ps[a + 1], m)
                      for a in range(0, len(ps), 2)]
                m *= 2
            out_v[pl.ds(i * B + g * L, L)] = -ps[0]
            return 0

        lax.fori_loop(0, NG, group, 0)

    issue(0, h0, t0, r0, s0)

    def pair(k, _):
        i = k * 2
        issue(i + 1, h1, t1, r1, s1)
        drain(h0, t0, r0, s0)
        compute(i, h0, t0, r0)
        issue(i + 2, h0, t0, r0, s0)
        drain(h1, t1, r1, s1)
        compute(i + 1, h1, t1, r1)
        return 0

    lax.fori_loop(0, (NCH - 1) // 2, pair, 0)
    drain(h0, t0, r0, s0)
    compute(NCH - 1, h0, t0, r0)
    pltpu.sync_copy(out_v, out_hbm.at[pl.ds(wid * EPW, EPW)])


@jax.jit
def _sc_score(znorm_p, rel_p, hidx, tidx, ridx):
    mesh = plsc.VectorSubcoreMesh(core_axis_name="c", subcore_axis_name="s",
                                  num_cores=NC, num_subcores=NS)
    return pl.kernel(
        _sc_body,
        out_type=jax.ShapeDtypeStruct((NUM_EDGES,), jnp.float32),
        mesh=mesh,
        compiler_params=pltpu.CompilerParams(needs_layout_passes=False,
                                             disable_bounds_checks=True,
                                             use_tc_tiling_on_sc=False),
        scratch_types=[
            pltpu.VMEM((EPW,), jnp.int32),
            pltpu.VMEM((EPW,), jnp.int32),
            pltpu.VMEM((EPW,), jnp.int32),
            pltpu.VMEM((B, HP), jnp.float32),
            pltpu.VMEM((B, HP), jnp.float32),
            pltpu.VMEM((B, HP), jnp.float32),
            pltpu.VMEM((B, HP), jnp.float32),
            pltpu.VMEM((B, HP), jnp.float32),
            pltpu.VMEM((B, HP), jnp.float32),
            pltpu.VMEM((EPW,), jnp.float32),
            pltpu.SemaphoreType.DMA,
            pltpu.SemaphoreType.DMA,
        ],
    )(znorm_p, rel_p, hidx, tidx, ridx)


def kernel(z, edge_index, edge_type, rel_emb):
    znorm_p = _pack_pairs(_l1_normalize_rows_bf16(z))
    rel_p = _pack_pairs(rel_emb.astype(jnp.bfloat16))
    hidx = edge_index[0].astype(jnp.int32)
    tidx = edge_index[1].astype(jnp.int32)
    ridx = edge_type.astype(jnp.int32)
    return _sc_score(znorm_p, rel_p, hidx, tidx, ridx)


# speedup vs baseline: 9.2954x; 1.4989x over previous
import jax
import jax.numpy as jnp
from jax import lax
from jax.experimental import pallas as pl
from jax.experimental.pallas import tpu as pltpu
from jax.experimental.pallas import tpu_sc as plsc

NUM_NODES = 10000
NUM_EDGES = 320000
NUM_RELATIONS = 1000
HIDDEN = 128

NC = 2
NS = 16
L = 16
NW = NC * NS
EPW = NUM_EDGES // NW
B = 80
NCH = EPW // B
NG = B // L
HP = HIDDEN // 2


def _norm_body(z_ref, o_ref):
    x = z_ref[...]
    n = jnp.sum(jnp.abs(x), axis=1, keepdims=True)
    o_ref[...] = (x / jnp.maximum(n, 1e-12)).astype(jnp.bfloat16)


def _l1_normalize_rows_bf16(z):
    return pl.pallas_call(
        _norm_body,
        out_shape=jax.ShapeDtypeStruct((NUM_NODES, HIDDEN), jnp.bfloat16),
        grid=(5,),
        in_specs=[pl.BlockSpec((NUM_NODES // 5, HIDDEN), lambda i: (i, 0))],
        out_specs=pl.BlockSpec((NUM_NODES // 5, HIDDEN), lambda i: (i, 0)),
    )(z)


def _pack_pairs(x_bf16):
    n, d = x_bf16.shape
    return lax.bitcast_convert_type(x_bf16.reshape(n, d // 2, 2), jnp.float32)


def _sc_body(znorm_hbm, rel_hbm, hidx_hbm, tidx_hbm, ridx_hbm, out_hbm,
             hidx_v, tidx_v, ridx_v, rel_v,
             h0, t0, h1, t1, out_v, s0, s1):
    wid = lax.axis_index("s") * NC + lax.axis_index("c")
    pltpu.sync_copy(hidx_hbm.at[pl.ds(wid * EPW, EPW)], hidx_v)
    pltpu.sync_copy(tidx_hbm.at[pl.ds(wid * EPW, EPW)], tidx_v)
    pltpu.sync_copy(ridx_hbm.at[pl.ds(wid * EPW, EPW)], ridx_v)
    pltpu.sync_copy(rel_hbm, rel_v)

    row16 = lax.iota(jnp.int32, L)

    def issue(i, hb, tb, sem):
        pltpu.async_copy(znorm_hbm.at[hidx_v.at[pl.ds(i * B, B)]], hb, sem)
        pltpu.async_copy(znorm_hbm.at[tidx_v.at[pl.ds(i * B, B)]], tb, sem)

    def drain(hb, tb, sem):
        pltpu.make_async_copy(znorm_hbm.at[pl.ds(0, B)], hb, sem).wait()
        pltpu.make_async_copy(znorm_hbm.at[pl.ds(0, B)], tb, sem).wait()

    def perm(v, m):
        return v.at[row16 ^ m].get(mode="promise_in_bounds")

    def combine(a, b, m):
        sa = a + perm(a, m)
        sb = b + perm(b, m)
        return jnp.where((row16 & m) == 0, sa, perm(sb, m))

    def compute(i, hb, tb):
        def group(g, _):
            rvec = ridx_v[pl.ds(i * B + g * L, L)]
            ps = []
            for j in range(L):
                e = g * L + j
                rid = rvec[j]
                acc_a = jnp.zeros((L,), jnp.float32)
                acc_b = jnp.zeros((L,), jnp.float32)
                for k in range(HP // L):
                    h = plsc.bitcast(hb[e, pl.ds(k * L, L)], jnp.bfloat16)
                    t = plsc.bitcast(tb[e, pl.ds(k * L, L)], jnp.bfloat16)
                    r = plsc.bitcast(rel_v[rid, pl.ds(k * L, L)], jnp.bfloat16)
                    v = jnp.abs(h + r - t)
                    va, vb = plsc.unpack(v, format=plsc.PackFormat.INTERLEAVED)
                    acc_a = acc_a + va
                    acc_b = acc_b + vb
                ps.append(acc_a + acc_b)
            m = 1
            while len(ps) > 1:
                ps = [combine(ps[a], ps[a + 1], m)
                      for a in range(0, len(ps), 2)]
                m *= 2
            out_v[pl.ds(i * B + g * L, L)] = -ps[0]
            return 0

        lax.fori_loop(0, NG, group, 0)

    issue(0, h0, t0, s0)

    def pair(k, _):
        i = k * 2
        issue(i + 1, h1, t1, s1)
        drain(h0, t0, s0)
        compute(i, h0, t0)
        issue(i + 2, h0, t0, s0)
        drain(h1, t1, s1)
        compute(i + 1, h1, t1)
        return 0

    lax.fori_loop(0, (NCH - 1) // 2, pair, 0)
    drain(h0, t0, s0)
    compute(NCH - 1, h0, t0)
    pltpu.sync_copy(out_v, out_hbm.at[pl.ds(wid * EPW, EPW)])


@jax.jit
def _sc_score(znorm_p, rel_p, hidx, tidx, ridx):
    mesh = plsc.VectorSubcoreMesh(core_axis_name="c", subcore_axis_name="s",
                                  num_cores=NC, num_subcores=NS)
    return pl.kernel(
        _sc_body,
        out_type=jax.ShapeDtypeStruct((NUM_EDGES,), jnp.float32),
        mesh=mesh,
        compiler_params=pltpu.CompilerParams(needs_layout_passes=False,
                                             disable_bounds_checks=True,
                                             use_tc_tiling_on_sc=False),
        scratch_types=[
            pltpu.VMEM((EPW,), jnp.int32),
            pltpu.VMEM((EPW,), jnp.int32),
            pltpu.VMEM((EPW,), jnp.int32),
            pltpu.VMEM((NUM_RELATIONS, HP), jnp.float32),
            pltpu.VMEM((B, HP), jnp.float32),
            pltpu.VMEM((B, HP), jnp.float32),
            pltpu.VMEM((B, HP), jnp.float32),
            pltpu.VMEM((B, HP), jnp.float32),
            pltpu.VMEM((EPW,), jnp.float32),
            pltpu.SemaphoreType.DMA,
            pltpu.SemaphoreType.DMA,
        ],
    )(znorm_p, rel_p, hidx, tidx, ridx)


def kernel(z, edge_index, edge_type, rel_emb):
    znorm_p = _pack_pairs(_l1_normalize_rows_bf16(z))
    rel_p = _pack_pairs(rel_emb.astype(jnp.bfloat16))
    hidx = edge_index[0].astype(jnp.int32)
    tidx = edge_index[1].astype(jnp.int32)
    ridx = edge_type.astype(jnp.int32)
    return _sc_score(znorm_p, rel_p, hidx, tidx, ridx)
